# baseline (speedup 1.0000x reference)
import jax
import jax.numpy as jnp
from jax import lax
from jax.experimental import pallas as pl
from jax.experimental.pallas import tpu as pltpu
from jax.experimental.pallas import tpu_sc as plsc

VOCAB = 1000000
EMBED = 32
MAXLEN = 200
BATCH = 4096
SEQ = 200
N = BATCH * SEQ
NC, NS = 2, 16
NW = NC * NS
PER_W = N // NW
FULL_TILES = VOCAB // 128
REM = VOCAB - FULL_TILES * 128
BASE_T = FULL_TILES // NW
EXTRA_T = FULL_TILES - BASE_T * NW


def _iota16():
    return lax.iota(jnp.int32, 16)


def _splat(v):
    return jnp.full((16,), v, dtype=jnp.int32)


def _relayout_body(tT, out1d, inA, inB, outA, outB, pin, pout,
                   gsA, gsB, wsA, wsB):
    cid = lax.axis_index("c")
    sid = lax.axis_index("s")
    w = sid * NC + cid
    base = w * BASE_T + jnp.minimum(w, EXTRA_T)
    R = BASE_T + jnp.where(w < EXTRA_T, 1, 0)
    RH = (R + 1) // 2

    def issue_read(t, buf, sem):
        pltpu.async_copy(tT.at[:, pl.ds(t * 128, 128)], buf, sem)

    def wait_read(buf, sem):
        pltpu.make_async_copy(tT.at[:, pl.ds(0, 128)], buf, sem).wait()

    def fire_write(t, buf, sem):
        pltpu.async_copy(buf, out1d.at[pl.ds(t * 4096, 4096)], sem)

    def drain_write(buf, sem):
        pltpu.make_async_copy(buf, out1d.at[pl.ds(0, 4096)], sem).wait()

    def transpose_tile(src, dst):
        def vh_body(vh, carry):
            for e in range(32):
                v = src[e, pl.ds(vh * 16, 16)]
                idx = _iota16() * 32 + (vh * 512 + e)
                plsc.store_scatter(dst, [idx], v)
            return carry
        lax.fori_loop(0, 8, vh_body, 0)

    issue_read(base, inA, gsA)

    def loop(i, carry):
        rA = 2 * i
        rB = 2 * i + 1
        @pl.when(rB < R)
        def _():
            issue_read(base + rB, inB, gsB)
        wait_read(inA, gsA)

        @pl.when(rA >= 2)
        def _():
            drain_write(outA, wsA)
        transpose_tile(inA, outA)
        fire_write(base + rA, outA, wsA)

        @pl.when(rB < R)
        def _():
            @pl.when(rB + 1 < R)
            def _():
                issue_read(base + rB + 1, inA, gsA)
            wait_read(inB, gsB)

            @pl.when(rB >= 3)
            def _():
                drain_write(outB, wsB)
            transpose_tile(inB, outB)
            fire_write(base + rB, outB, wsB)
        return carry

    lax.fori_loop(0, RH, loop, 0)
    drain_write(outA, wsA)
    drain_write(outB, wsB)

    @pl.when(w == NW - 1)
    def _():
        pltpu.sync_copy(tT.at[:, pl.ds(FULL_TILES * 128, REM)], pin)

        def vh_body(vh, carry):
            for e in range(32):
                v = pin[e, pl.ds(vh * 16, 16)]
                idx = _iota16() * 32 + (vh * 512 + e)
                plsc.store_scatter(pout, [idx], v)
            return carry
        lax.fori_loop(0, REM // 16, vh_body, 0)
        pltpu.sync_copy(pout, out1d.at[pl.ds(FULL_TILES * 4096, REM * 32)])


def _gather_body(xf, lin, pos, out1, x_v, idxT, pos_v, rowsA, rowsB,
                 obufA, obufB, gsA, gsB, wsA, wsB):
    cid = lax.axis_index("c")
    sid = lax.axis_index("s")
    w = sid * NC + cid

    pltpu.sync_copy(xf.at[pl.ds(w * PER_W, PER_W)], x_v.at[pl.ds(0, PER_W)])
    pltpu.sync_copy(pos, pos_v)

    tail_mask = _iota16() < (SEQ - (SEQ // 16) * 16)

    def build(b, carry):
        for lh in range(SEQ // 16):
            v = x_v[pl.ds(b * SEQ + lh * 16, 16)]
            idx = _iota16() * 128 + (lh * 16 * 128 + b)
            plsc.store_scatter(idxT, [idx], v)
        lh = SEQ // 16
        v = x_v[pl.ds(b * SEQ + lh * 16, 16)]
        idx = _iota16() * 128 + (lh * 16 * 128 + b)
        plsc.store_scatter(idxT, [idx], v, mask=tail_mask)
        return carry

    lax.fori_loop(0, 128, build, 0)

    def issue_gather(l, rows, sem):
        pltpu.async_copy(lin.at[idxT.at[pl.ds(l * 128, 128)]], rows, sem)

    def wait_gather(rows, sem):
        pltpu.make_async_copy(
            lin.at[idxT.at[pl.ds(0, 128)]], rows, sem).wait()

    def fire_write(l, obuf, sem):
        for eg in range(4):
            pltpu.async_copy(
                obuf.at[pl.ds(eg * 1024, 1024)],
                out1.at[pl.ds(l * 131072 + eg * 32768 + w * 1024, 1024)],
                sem)

    def drain_write(obuf, sem):
        for eg in range(4):
            pltpu.make_async_copy(
                obuf.at[pl.ds(eg * 1024, 1024)],
                out1.at[pl.ds(0, 1024)], sem).wait()

    def compute(l, rows, obuf):
        af0 = pos_v[l, pl.ds(0, 16)]
        af1 = pos_v[l, pl.ds(16, 16)]

        def bb_body(bb, carry):
            for j in range(8):
                b = bb * 8 + j
                v0 = rows[b, pl.ds(0, 16)] + af0
                v1 = rows[b, pl.ds(16, 16)] + af1
                plsc.store_scatter(obuf, [_iota16() * 128 + b], v0)
                plsc.store_scatter(obuf, [_iota16() * 128 + (b + 2048)], v1)
            return carry

        lax.fori_loop(0, 16, bb_body, 0)

    issue_gather(0, rowsA, gsA)

    def loop(i, carry):
        lA = 2 * i
        lB = 2 * i + 1
        issue_gather(lB, rowsB, gsB)
        wait_gather(rowsA, gsA)

        @pl.when(i > 0)
        def _():
            drain_write(obufA, wsA)
        compute(lA, rowsA, obufA)
        fire_write(lA, obufA, wsA)

        @pl.when(i < SEQ // 2 - 1)
        def _():
            issue_gather(lB + 1, rowsA, gsA)

        wait_gather(rowsB, gsB)

        @pl.when(i > 0)
        def _():
            drain_write(obufB, wsB)
        compute(lB, rowsB, obufB)
        fire_write(lB, obufB, wsB)
        return carry

    lax.fori_loop(0, SEQ // 2, loop, 0)
    drain_write(obufA, wsA)
    drain_write(obufB, wsB)


def kernel(x, table, pos_table):
    tT = table.T

    k1 = pl.kernel(
        _relayout_body,
        out_type=jax.ShapeDtypeStruct((VOCAB * EMBED,), jnp.float32),
        mesh=plsc.VectorSubcoreMesh(core_axis_name="c", subcore_axis_name="s"),
        compiler_params=pltpu.CompilerParams(
            use_tc_tiling_on_sc=True, needs_layout_passes=False),
        scratch_types=[
            pltpu.VMEM((32, 128), jnp.float32),
            pltpu.VMEM((32, 128), jnp.float32),
            pltpu.VMEM((4096,), jnp.float32),
            pltpu.VMEM((4096,), jnp.float32),
            pltpu.VMEM((32, REM), jnp.float32),
            pltpu.VMEM((REM * 32,), jnp.float32),
            pltpu.SemaphoreType.DMA,
            pltpu.SemaphoreType.DMA,
            pltpu.SemaphoreType.DMA,
            pltpu.SemaphoreType.DMA,
        ],
    )
    lin = k1(tT).reshape(VOCAB, EMBED)

    xf = x.reshape(N)
    k2 = pl.kernel(
        _gather_body,
        out_type=jax.ShapeDtypeStruct((SEQ * 4 * 32 * 8 * 128,), jnp.float32),
        mesh=plsc.VectorSubcoreMesh(core_axis_name="c", subcore_axis_name="s"),
        compiler_params=pltpu.CompilerParams(
            use_tc_tiling_on_sc=False, needs_layout_passes=False),
        scratch_types=[
            pltpu.VMEM((PER_W + 16,), jnp.int32),
            pltpu.VMEM((PER_W,), jnp.int32),
            pltpu.VMEM((MAXLEN, EMBED), jnp.float32),
            pltpu.VMEM((128, EMBED), jnp.float32),
            pltpu.VMEM((128, EMBED), jnp.float32),
            pltpu.VMEM((4096,), jnp.float32),
            pltpu.VMEM((4096,), jnp.float32),
            pltpu.SemaphoreType.DMA,
            pltpu.SemaphoreType.DMA,
            pltpu.SemaphoreType.DMA,
            pltpu.SemaphoreType.DMA,
        ],
    )
    out1 = k2(xf, lin, pos_table)
    out = (out1.reshape(SEQ, 4, 32, 8, 128)
           .transpose(2, 4, 0, 1, 3)
           .reshape(BATCH, SEQ, EMBED))
    return out
